# baseline (speedup 1.0000x reference)
import functools

import jax
import jax.numpy as jnp
from jax import lax
from jax.experimental import pallas as pl
from jax.experimental.pallas import tpu as pltpu
from jax.experimental.pallas import tpu_sc as plsc

_CHUNK = 128
_W = 128


@functools.lru_cache(maxsize=None)
def _build(B, D):
    info = plsc.get_sparse_core_info()
    NC, NS = info.num_cores, info.num_subcores
    NW = NC * NS
    b_per_w = B // NW
    assert B % (NW * _CHUNK) == 0 and D % 16 == 0
    cpw = b_per_w // _CHUNK

    mesh = plsc.VectorSubcoreMesh(core_axis_name="c", subcore_axis_name="s")

    @functools.partial(
        pl.kernel,
        mesh=mesh,
        out_type=jax.ShapeDtypeStruct((B, 2 * D), jnp.float32),
        scratch_types=[
            pltpu.VMEM((cpw, _CHUNK), jnp.int32),
            pltpu.VMEM((cpw, _CHUNK), jnp.int32),
            pltpu.VMEM((_CHUNK, _W), jnp.float32),
            pltpu.VMEM((_CHUNK, _W), jnp.float32),
            pltpu.VMEM((_CHUNK, _W), jnp.float32),
            pltpu.VMEM((_CHUNK, _W), jnp.float32),
            pltpu.VMEM((_CHUNK, 2 * D), jnp.float32),
            pltpu.VMEM((_CHUNK, 2 * D), jnp.float32),
            pltpu.SemaphoreType.DMA,
            pltpu.SemaphoreType.DMA,
        ],
    )
    def k(uids_hbm, bids_hbm, utab_hbm, btab_hbm, out_hbm,
          uidx_v, bidx_v, u_v0, u_v1, b_v0, b_v1, comb_v0, comb_v1,
          gsem, osem):
        u_ring, b_ring, comb_ring = (u_v0, u_v1), (b_v0, b_v1), (comb_v0, comb_v1)
        wid = lax.axis_index("s") * NC + lax.axis_index("c")
        base = wid * b_per_w
        pltpu.sync_copy(uids_hbm.at[pl.ds(wid * cpw, cpw)], uidx_v)
        pltpu.sync_copy(bids_hbm.at[pl.ds(wid * cpw, cpw)], bidx_v)

        def fire(j):
            s = j % 2
            cu = pltpu.async_copy(utab_hbm.at[uidx_v.at[j]], u_ring[s], gsem)
            cb = pltpu.async_copy(btab_hbm.at[bidx_v.at[j]], b_ring[s], gsem)
            return cu, cb

        def interleave(s):
            u_v, b_v, comb_v = u_ring[s], b_ring[s], comb_ring[s]

            def body(i, _):
                for c in range(D // 16):
                    comb_v[i, pl.ds(16 * c, 16)] = u_v[i, pl.ds(16 * c, 16)]
                    comb_v[i, pl.ds(D + 16 * c, 16)] = b_v[i, pl.ds(16 * c, 16)]
                return 0
            lax.fori_loop(0, _CHUNK, body, 0)

        pending = fire(0)
        out_cp = None
        for j in range(cpw):
            s = j % 2
            for c in pending:
                c.wait()
            if j + 1 < cpw:
                pending = fire(j + 1)
            interleave(s)
            if out_cp is not None:
                out_cp.wait()
            out_cp = pltpu.async_copy(
                comb_ring[s], out_hbm.at[pl.ds(base + j * _CHUNK, _CHUNK)],
                osem)
        out_cp.wait()

    return k


def kernel(user_ids, book_ids, user_table, book_table):
    B = user_ids.shape[0]
    V, D = user_table.shape
    uids = user_ids.astype(jnp.int32).reshape(B // _CHUNK, _CHUNK)
    bids = book_ids.astype(jnp.int32).reshape(B // _CHUNK, _CHUNK)
    utab = jnp.pad(user_table, ((0, 0), (0, _W - D)))
    btab = jnp.pad(book_table, ((0, 0), (0, _W - D)))
    k = _build(B, D)
    return k(uids, bids, utab, btab)
